# baseline (speedup 1.0000x reference)
import functools

import jax
import jax.numpy as jnp
from jax import lax
from jax.experimental import pallas as pl
from jax.experimental.pallas import tpu as pltpu
from jax.experimental.pallas import tpu_sc as plsc

NUM_CORES = 2
NUM_SUBCORES = 16
NUM_WORKERS = NUM_CORES * NUM_SUBCORES
LANES = 16

NUM_ROWS = 1000000
BATCH = 16384
DIM = 32
B_PER_W = BATCH // NUM_WORKERS
GROUPS = B_PER_W // LANES
N_ELEMS = B_PER_W * DIM


def _mf_body(uidx_hbm, iidx_hbm, ut_hbm, it_hbm, w_hbm, out_hbm,
             idx_u, idx_i, ids_u, ids_i, g_u, g_i, w_v, out_v, sem):
    wid = lax.axis_index("s") * NUM_CORES + lax.axis_index("c")
    base = wid * B_PER_W

    pltpu.sync_copy(uidx_hbm.at[wid], idx_u)
    pltpu.sync_copy(iidx_hbm.at[wid], idx_i)
    pltpu.sync_copy(w_hbm, w_v)

    def build_body(g, carry):
        u16 = idx_u[pl.ds(g * LANES, LANES)]
        i16 = idx_i[pl.ds(g * LANES, LANES)]
        for d in range(DIM):
            off = jnp.full((LANES,), d * NUM_ROWS, jnp.int32)
            ids_u[pl.ds(d * B_PER_W + g * LANES, LANES)] = u16 + off
            ids_i[pl.ds(d * B_PER_W + g * LANES, LANES)] = i16 + off
        return carry

    lax.fori_loop(0, GROUPS, build_body, 0)

    cu = pltpu.async_copy(ut_hbm.at[ids_u], g_u, sem)
    ci = pltpu.async_copy(it_hbm.at[ids_i], g_i, sem)
    cu.wait()
    ci.wait()

    def group_body(g, carry):
        acc = jnp.zeros((LANES,), jnp.float32)
        for d in range(DIM):
            off = d * B_PER_W + g * LANES
            acc = acc + (g_u[pl.ds(off, LANES)] * g_i[pl.ds(off, LANES)]
                         * w_v[d, :])
        sig = 1.0 / (1.0 + jnp.exp(-acc))
        out_v[pl.ds(g * LANES, LANES)] = sig
        return carry

    lax.fori_loop(0, GROUPS, group_body, 0)

    pltpu.sync_copy(out_v, out_hbm.at[pl.ds(base, B_PER_W)])


@functools.partial(
    pl.kernel,
    out_type=jax.ShapeDtypeStruct((BATCH,), jnp.float32),
    mesh=plsc.VectorSubcoreMesh(core_axis_name="c", subcore_axis_name="s"),
    scratch_types=[
        pltpu.VMEM((B_PER_W,), jnp.int32),
        pltpu.VMEM((B_PER_W,), jnp.int32),
        pltpu.VMEM((N_ELEMS,), jnp.int32),
        pltpu.VMEM((N_ELEMS,), jnp.int32),
        pltpu.VMEM((N_ELEMS,), jnp.float32),
        pltpu.VMEM((N_ELEMS,), jnp.float32),
        pltpu.VMEM((DIM, LANES), jnp.float32),
        pltpu.VMEM((B_PER_W,), jnp.float32),
        pltpu.SemaphoreType.DMA,
    ],
    compiler_params=pltpu.CompilerParams(
        needs_layout_passes=False, use_tc_tiling_on_sc=False),
)
def _mf_kernel(*refs):
    _mf_body(*refs)


def kernel(user_indices, item_indices, user_emb, item_emb, fc_w):
    uidx = user_indices.astype(jnp.int32).reshape(NUM_WORKERS, B_PER_W)
    iidx = item_indices.astype(jnp.int32).reshape(NUM_WORKERS, B_PER_W)
    ut = user_emb.T.reshape(NUM_ROWS * DIM)
    it = item_emb.T.reshape(NUM_ROWS * DIM)
    w_b = jnp.broadcast_to(fc_w.reshape(DIM, 1), (DIM, LANES))
    return _mf_kernel(uidx, iidx, ut, it, w_b)
